# Initial kernel scaffold; baseline (speedup 1.0000x reference)
import jax
import jax.numpy as jnp
from jax.experimental import pallas as pl


def kernel(entity_emb, cls_embeddings, triple_heads, triple_indices, target_head_id, W, b):
    raise NotImplementedError("write your pallas kernel here")


# speedup vs baseline: 1.0436x; 1.0436x over previous
import functools

import jax
import jax.numpy as jnp
from jax import lax
from jax.experimental import pallas as pl
from jax.experimental.pallas import tpu as pltpu
from jax.experimental.pallas import tpu_sc as plsc

N_SEL = 8192
D = 256
NC, NS, L = 2, 16, 16
NW = NC * NS
CHUNK = N_SEL // NW
IB = 128


def _sc_stage(triple_heads, triple_indices, cls_embeddings, tgt_vec, entity_emb):
    mesh = plsc.VectorSubcoreMesh(core_axis_name="c", subcore_axis_name="s")

    @functools.partial(
        pl.kernel,
        out_type=[
            jax.ShapeDtypeStruct((NW, D), jnp.float32),
            jax.ShapeDtypeStruct((NW, L), jnp.int32),
            jax.ShapeDtypeStruct((1, D), jnp.float32),
        ],
        mesh=mesh,
        scratch_types=[
            pltpu.VMEM((CHUNK // IB, IB), jnp.int32),
            pltpu.VMEM((CHUNK // IB, IB), jnp.int32),
            pltpu.VMEM((CHUNK,), jnp.int32),
            pltpu.VMEM((L, D), jnp.float32),
            pltpu.VMEM((D,), jnp.float32),
            pltpu.VMEM((L,), jnp.int32),
            pltpu.VMEM((L,), jnp.int32),
            pltpu.VMEM((1, D), jnp.float32),
            pltpu.SemaphoreType.DMA,
        ],
    )
    def k(th_hbm, ti_hbm, cls_hbm, tgt_hbm, ent_hbm,
          part_out, cnt_out, ent_out,
          idx_v, heads_v, w_v, rows_v, acc_v, tgt_v, cnt_v, entrow_v, sem):
        c = lax.axis_index("c")
        s = lax.axis_index("s")
        wid = s * NC + c
        base = wid * CHUNK

        pltpu.sync_copy(tgt_hbm, tgt_v)
        tgtv = tgt_v[...]

        for j in range(CHUNK // IB):
            pltpu.sync_copy(ti_hbm.at[pl.ds(base + j * IB, IB)], idx_v.at[j])
        for j in range(CHUNK // IB):
            pltpu.async_copy(th_hbm.at[idx_v.at[j]], heads_v.at[j], sem).wait()

        zf = jnp.zeros((L,), jnp.float32)
        for j in range(D // L):
            acc_v[pl.ds(j * L, L)] = zf

        for ci in range(CHUNK // L):
            hv = heads_v[ci // (IB // L), pl.ds((ci % (IB // L)) * L, L)]
            m = hv == tgtv
            w_v[pl.ds(ci * L, L)] = jnp.where(m, 1, 0)

        def bbody(ci, tot):
            wv = w_v[pl.ds(ci * L, L)]
            ws = [wv[r] for r in range(L)]
            msum = ws[0]
            for r in range(1, L):
                msum = msum + ws[r]

            @pl.when(msum > 0)
            def _():
                pltpu.sync_copy(cls_hbm.at[pl.ds(base + ci * L, L)], rows_v)
                wf = [jnp.where(w > 0, 1.0, 0.0) for w in ws]
                for dc in range(D // L):
                    a = acc_v[pl.ds(dc * L, L)]
                    for r in range(L):
                        a = a + rows_v[r, pl.ds(dc * L, L)] * wf[r]
                    acc_v[pl.ds(dc * L, L)] = a

            return tot + msum

        cnt = lax.fori_loop(0, CHUNK // L, bbody, jnp.int32(0))

        pltpu.sync_copy(acc_v, part_out.at[wid])
        cnt_v[...] = jnp.broadcast_to(cnt, (L,))
        pltpu.sync_copy(cnt_v, cnt_out.at[wid])

        @pl.when(wid == 0)
        def _():
            t_s = tgtv[0]
            pltpu.sync_copy(ent_hbm.at[pl.ds(t_s, 1)], entrow_v)
            pltpu.sync_copy(entrow_v, ent_out)

    return k(triple_heads, triple_indices, cls_embeddings, tgt_vec, entity_emb)


def _tc_stage(part, cnts, entrow, W, b):
    def body(part_ref, cnt_ref, ent_ref, w_ref, b_ref, out_ref):
        summed = jnp.sum(part_ref[...], axis=0)
        cnt_f = jnp.sum(cnt_ref[...].astype(jnp.float32)) / L
        mean = summed / jnp.maximum(cnt_f, 1.0)
        upd = lax.dot_general(mean[None, :], w_ref[...],
                              (((1,), (1,)), ((), ())),
                              preferred_element_type=jnp.float32)[0] + b_ref[...]
        out_ref[...] = jnp.where(cnt_f > 0.0, upd, ent_ref[0])

    return pl.pallas_call(
        body,
        out_shape=jax.ShapeDtypeStruct((D,), jnp.float32),
    )(part, cnts, entrow, W, b)


def kernel(entity_emb, cls_embeddings, triple_heads, triple_indices, target_head_id, W, b):
    tgt_vec = jnp.broadcast_to(jnp.asarray(target_head_id, jnp.int32), (L,))
    part, cnts, entrow = _sc_stage(
        triple_heads, triple_indices, cls_embeddings, tgt_vec, entity_emb)
    return _tc_stage(part, cnts, entrow, W, b)
